# baseline (speedup 1.0000x reference)
import dataclasses
import functools

import jax
import jax.numpy as jnp
from jax.experimental import pallas as pl
from jax.experimental.pallas import tpu as pltpu
from jax.experimental.pallas import tpu_sc as plsc

_X_LOW = -4.0
_X_HIGH = 4.0
_N = 1024
_MULT = _N / (_X_HIGH - _X_LOW)
_ADD = _X_LOW * _N / (_X_LOW - _X_HIGH)

_LANES = 16
_COLS = 2048
_BROWS = 8


def kernel(x, table):
    rows = x.size // _COLS
    x2d = x.reshape(rows, _COLS)
    mesh = plsc.VectorSubcoreMesh(core_axis_name="c", subcore_axis_name="s")
    cp = pltpu.CompilerParams(use_tc_tiling_on_sc=True)
    if "needs_layout_passes" in pltpu.CompilerParams.__dataclass_fields__:
        cp = dataclasses.replace(cp, needs_layout_passes=False)

    @functools.partial(
        pl.kernel,
        out_type=jax.ShapeDtypeStruct((rows, _COLS), jnp.float32),
        mesh=mesh,
        scratch_types=[pltpu.VMEM((_N,), jnp.float32)],
        compiler_params=cp,
    )
    def pac(x_hbm, t_hbm, o_hbm, t_vmem):
        pltpu.sync_copy(t_hbm, t_vmem)

        def body(in_v, out_v):
            @plsc.parallel_loop(0, _COLS, step=_LANES, unroll=4)
            def _(c):
                for r in range(_BROWS):
                    sl = (r, pl.ds(c, _LANES))
                    f = in_v[sl] * _MULT + _ADD
                    f = jnp.minimum(jnp.maximum(f, 0.0), float(_N - 1))
                    idx = f.astype(jnp.int32)
                    out_v[sl] = plsc.load_gather(t_vmem, [idx])

        pltpu.emit_pipeline(
            body,
            grid=(rows // _BROWS,),
            in_specs=[pl.BlockSpec((_BROWS, _COLS), lambda i: (i, 0))],
            out_specs=[pl.BlockSpec((_BROWS, _COLS), lambda i: (i, 0))],
            core_axis_name=("c", "s"),
            dimension_semantics=(pltpu.PARALLEL,),
        )(x_hbm, o_hbm)

    return pac(x2d, table).reshape(x.shape)


# speedup vs baseline: 1.0482x; 1.0482x over previous
import dataclasses
import functools

import jax
import jax.numpy as jnp
from jax.experimental import pallas as pl
from jax.experimental.pallas import tpu as pltpu
from jax.experimental.pallas import tpu_sc as plsc

_X_LOW = -4.0
_X_HIGH = 4.0
_N = 1024
_MULT = _N / (_X_HIGH - _X_LOW)
_ADD = _X_LOW * _N / (_X_LOW - _X_HIGH)

_LANES = 16
_COLS = 2048
_BROWS = 8


def kernel(x, table):
    rows = x.size // _COLS
    x2d = x.reshape(rows, _COLS)
    mesh = plsc.VectorSubcoreMesh(core_axis_name="c", subcore_axis_name="s")
    cp = pltpu.CompilerParams(use_tc_tiling_on_sc=True)
    if "needs_layout_passes" in pltpu.CompilerParams.__dataclass_fields__:
        cp = dataclasses.replace(cp, needs_layout_passes=False)

    @functools.partial(
        pl.kernel,
        out_type=jax.ShapeDtypeStruct((rows, _COLS), jnp.float32),
        mesh=mesh,
        scratch_types=[pltpu.VMEM((_N,), jnp.float32)],
        compiler_params=cp,
    )
    def pac(x_hbm, t_hbm, o_hbm, t_vmem):
        pltpu.sync_copy(t_hbm, t_vmem)

        def body(in_v, out_v):
            @plsc.parallel_loop(0, _COLS, step=_LANES, unroll=8)
            def _(c):
                for r in range(_BROWS):
                    sl = (r, pl.ds(c, _LANES))
                    f = in_v[sl] * _MULT + _ADD
                    f = jnp.minimum(jnp.maximum(f, 0.0), float(_N - 1))
                    idx = f.astype(jnp.int32)
                    out_v[sl] = plsc.load_gather(t_vmem, [idx])

        pltpu.emit_pipeline(
            body,
            grid=(rows // _BROWS,),
            in_specs=[pl.BlockSpec((_BROWS, _COLS), lambda i: (i, 0))],
            out_specs=[pl.BlockSpec((_BROWS, _COLS), lambda i: (i, 0))],
            core_axis_name=("c", "s"),
            dimension_semantics=(pltpu.PARALLEL,),
        )(x_hbm, o_hbm)

    return pac(x2d, table).reshape(x.shape)
